# baseline (device time: 119687 ns/iter reference)
import jax
import jax.numpy as jnp
from jax import lax
from jax.experimental import pallas as pl
from jax.experimental.pallas import tpu as pltpu

N_DEV = 8


def kernel(x, router_W, route_idx, expert_W):
    del router_W
    n_tok, d_model = x.shape
    n_local_e, _, d_ff = expert_W.shape
    C = n_tok // N_DEV

    def body(x_ref, idx_ref, w_ref, out_ref, acc_ref, rs_ref,
             rs_send, rs_recv, ag_send, ag_recv):
        my = lax.axis_index("i")
        left = lax.rem(my - 1 + N_DEV, N_DEV)
        right = lax.rem(my + 1, N_DEV)

        barrier_sem = pltpu.get_barrier_semaphore()
        for nbr in [left, right]:
            pl.semaphore_signal(
                barrier_sem, inc=1,
                device_id=(nbr,), device_id_type=pl.DeviceIdType.MESH,
            )
        pl.semaphore_wait(barrier_sem, 2)

        total = None
        for e in range(n_local_e):
            ge = my * n_local_e + e
            m = (idx_ref[:, :] == ge).astype(jnp.float32)
            contrib = jnp.dot(
                x_ref[:, :] * m, w_ref[e, :, :],
                preferred_element_type=jnp.float32,
            )
            total = contrib if total is None else total + contrib
        acc_ref[:, :] = total

        for s in range(N_DEV - 1):
            send_chunk = lax.rem(my - s + N_DEV, N_DEV)
            if s == 0:
                src = acc_ref.at[pl.ds(send_chunk * C, C), :]
            else:
                src = rs_ref.at[s - 1]
            rdma = pltpu.make_async_remote_copy(
                src_ref=src,
                dst_ref=rs_ref.at[s],
                send_sem=rs_send.at[s],
                recv_sem=rs_recv.at[s],
                device_id=(right,),
                device_id_type=pl.DeviceIdType.MESH,
            )
            rdma.start()
            rdma.wait()
            r = lax.rem(my - s - 1 + N_DEV, N_DEV)
            rs_ref[s, :, :] = rs_ref[s, :, :] + acc_ref[pl.ds(r * C, C), :]

        own = lax.rem(my + 1, N_DEV)
        out_ref[pl.ds(own * C, C), :] = rs_ref[N_DEV - 2, :, :]

        for t in range(N_DEV - 1):
            sc = lax.rem(own - t + N_DEV, N_DEV)
            rdma = pltpu.make_async_remote_copy(
                src_ref=out_ref.at[pl.ds(sc * C, C), :],
                dst_ref=out_ref.at[pl.ds(sc * C, C), :],
                send_sem=ag_send.at[t],
                recv_sem=ag_recv.at[t],
                device_id=(right,),
                device_id_type=pl.DeviceIdType.MESH,
            )
            rdma.start()
            rdma.wait()

    return pl.pallas_call(
        body,
        out_shape=jax.ShapeDtypeStruct((n_tok, d_ff), jnp.float32),
        in_specs=[
            pl.BlockSpec(memory_space=pltpu.VMEM),
            pl.BlockSpec(memory_space=pltpu.VMEM),
            pl.BlockSpec(memory_space=pltpu.VMEM),
        ],
        out_specs=pl.BlockSpec(memory_space=pltpu.VMEM),
        scratch_shapes=[
            pltpu.VMEM((n_tok, d_ff), jnp.float32),
            pltpu.VMEM((N_DEV - 1, C, d_ff), jnp.float32),
            pltpu.SemaphoreType.DMA((N_DEV - 1,)),
            pltpu.SemaphoreType.DMA((N_DEV - 1,)),
            pltpu.SemaphoreType.DMA((N_DEV - 1,)),
            pltpu.SemaphoreType.DMA((N_DEV - 1,)),
        ],
        compiler_params=pltpu.CompilerParams(collective_id=0),
    )(x, route_idx, expert_W)


# device time: 57401 ns/iter; 2.0851x vs baseline; 2.0851x over previous
import jax
import jax.numpy as jnp
from jax import lax
from jax.experimental import pallas as pl
from jax.experimental.pallas import tpu as pltpu

N_DEV = 8

PARTS = ((0, 384, (0, 1, 2)), (384, 384, (1, 2, 0)), (768, 256, (2, 0, 1)))
BIT_MASK = (1, 3, 4)
RS_BASE = (0, 4, 6)
RS_COUNT = (4, 2, 1)
AG_BASE = (0, 1, 3)
AG_COUNT = (1, 2, 4)


def kernel(x, router_W, route_idx, expert_W):
    del router_W
    n_tok, d_model = x.shape
    n_local_e, _, d_ff = expert_W.shape
    C = n_tok // N_DEV

    def body(x_ref, idx_ref, w_ref, out_ref, acc_ref, rsb_ref,
             rs_send, rs_recv, ag_send, ag_recv):
        my = lax.axis_index("i")
        b0 = my & 1
        b1 = (my >> 1) & 1
        b2 = (my >> 2) & 1
        a = (b0 ^ b1, b1, b2)
        partner = tuple(my ^ m for m in BIT_MASK)

        barrier_sem = pltpu.get_barrier_semaphore()
        for p in partner:
            pl.semaphore_signal(
                barrier_sem, inc=1,
                device_id=(p,), device_id_type=pl.DeviceIdType.MESH,
            )
        pl.semaphore_wait(barrier_sem, 3)

        total = None
        for e in range(n_local_e):
            ge = my * n_local_e + e
            m = (idx_ref[:, :] == ge).astype(jnp.float32)
            contrib = jnp.dot(
                x_ref[:, :] * m, w_ref[e, :, :],
                preferred_element_type=jnp.float32,
            )
            total = contrib if total is None else total + contrib
        acc_ref[:, :] = total

        def chunk_id(order, k, mine, j, n_fixed):
            c = 0
            for i in range(n_fixed):
                if order[i] != order[k]:
                    c += a[order[i]] * (1 << order[i])
            bit = a[order[k]] if mine else 1 - a[order[k]]
            c += bit * (1 << order[k])
            free = [b for i, b in enumerate(order) if i >= n_fixed and b != order[k]]
            for i, fb in enumerate(free):
                c += ((j >> i) & 1) * (1 << fb)
            return c

        for k in range(3):
            started = []
            for pi, (c0, w, order) in enumerate(PARTS):
                bk = order[k]
                for j in range(RS_COUNT[k]):
                    cs = chunk_id(order, k, False, j, k)
                    cr = chunk_id(order, k, True, j, k)
                    slot = RS_BASE[k] + j
                    rdma = pltpu.make_async_remote_copy(
                        src_ref=acc_ref.at[pl.ds(cs * C, C), pl.ds(c0, w)],
                        dst_ref=rsb_ref.at[slot, :, pl.ds(c0, w)],
                        send_sem=rs_send.at[pi, slot],
                        recv_sem=rs_recv.at[pi, slot],
                        device_id=(partner[bk],),
                        device_id_type=pl.DeviceIdType.MESH,
                    )
                    rdma.start()
                    started.append((rdma, slot, cr, c0, w))
            for rdma, slot, cr, c0, w in started:
                rdma.wait()
                acc_ref[pl.ds(cr * C, C), pl.ds(c0, w)] = (
                    acc_ref[pl.ds(cr * C, C), pl.ds(c0, w)]
                    + rsb_ref[slot, :, pl.ds(c0, w)]
                )

        c_own = a[0] + 2 * a[1] + 4 * a[2]
        out_ref[pl.ds(c_own * C, C), :] = acc_ref[pl.ds(c_own * C, C), :]

        for k in range(3):
            started = []
            for pi, (c0, w, order) in enumerate(PARTS):
                bk = order[2 - k]
                for j in range(AG_COUNT[k]):
                    cs = chunk_id(order, 2 - k, True, j, 2 - k)
                    slot = AG_BASE[k] + j
                    rdma = pltpu.make_async_remote_copy(
                        src_ref=out_ref.at[pl.ds(cs * C, C), pl.ds(c0, w)],
                        dst_ref=out_ref.at[pl.ds(cs * C, C), pl.ds(c0, w)],
                        send_sem=ag_send.at[pi, slot],
                        recv_sem=ag_recv.at[pi, slot],
                        device_id=(partner[bk],),
                        device_id_type=pl.DeviceIdType.MESH,
                    )
                    rdma.start()
                    started.append(rdma)
            for rdma in started:
                rdma.wait()

    return pl.pallas_call(
        body,
        out_shape=jax.ShapeDtypeStruct((n_tok, d_ff), jnp.float32),
        in_specs=[
            pl.BlockSpec(memory_space=pltpu.VMEM),
            pl.BlockSpec(memory_space=pltpu.VMEM),
            pl.BlockSpec(memory_space=pltpu.VMEM),
        ],
        out_specs=pl.BlockSpec(memory_space=pltpu.VMEM),
        scratch_shapes=[
            pltpu.VMEM((n_tok, d_ff), jnp.float32),
            pltpu.VMEM((N_DEV - 1, C, d_ff), jnp.float32),
            pltpu.SemaphoreType.DMA((len(PARTS), 7)),
            pltpu.SemaphoreType.DMA((len(PARTS), 7)),
            pltpu.SemaphoreType.DMA((len(PARTS), 7)),
            pltpu.SemaphoreType.DMA((len(PARTS), 7)),
        ],
        compiler_params=pltpu.CompilerParams(collective_id=0),
    )(x, route_idx, expert_W)
